# baseline (speedup 1.0000x reference)
import functools

import jax
import jax.numpy as jnp
from jax import lax
from jax.experimental import pallas as pl
from jax.experimental.pallas import tpu as pltpu
from jax.experimental.pallas import tpu_sc as plsc

_N = 20000
_B = 1024
_K = 20
_NP = _B * _K
_KL = 128
_BC = 256
_NC = _B // _BC


def _nms_body(scal_ref, yolo_ref, yoloT_ref, out_ref, d_ref, dcol_ref,
              sup_ref, comp_ref, nch_ref):
    iou_thr = scal_ref[0]
    score_thr = scal_ref[1]

    cx = yolo_ref[0]
    cy = yolo_ref[1]
    w = yolo_ref[2]
    h = yolo_ref[3]
    s = yolo_ref[4]
    x1 = cx - w / 2
    y1 = cy - h / 2
    x2 = cx + w / 2
    y2 = cy + h / 2
    area = jnp.maximum(x2 - x1, 0.0) * jnp.maximum(y2 - y1, 0.0)
    d_ref[0] = x1
    d_ref[1] = y1
    d_ref[2] = x2
    d_ref[3] = y2
    d_ref[4] = area

    cxT = yoloT_ref[0]
    cyT = yoloT_ref[1]
    wT = yoloT_ref[2]
    hT = yoloT_ref[3]
    x1T = cxT - wT / 2
    y1T = cyT - hT / 2
    x2T = cxT + wT / 2
    y2T = cyT + hT / 2
    dcol_ref[0] = x1T
    dcol_ref[1] = y1T
    dcol_ref[2] = x2T
    dcol_ref[3] = y2T
    dcol_ref[4] = jnp.maximum(x2T - x1T, 0.0) * jnp.maximum(y2T - y1T, 0.0)

    sup_ref[...] = (s <= score_thr).astype(jnp.float32)

    n_eff = jnp.sum((s > score_thr).astype(jnp.int32))
    k_eff = (n_eff + _B - 1) // _B

    ri = lax.broadcasted_iota(jnp.int32, (_B, _B), 0)
    ci = lax.broadcasted_iota(jnp.int32, (_B, _B), 1)
    tri = (ci > ri).astype(jnp.float32)
    ident = (ci == ri).astype(jnp.float32)
    lowtri = (ci < ri).astype(jnp.float32)
    qf = lax.broadcasted_iota(jnp.int32, (_B, _BC), 1).astype(jnp.float32)
    lane_k = lax.broadcasted_iota(jnp.int32, (1, _KL), 1)

    def cols_of(m):
        onehot = (lane_k == m).astype(jnp.float32)
        return [jnp.sum(dcol_ref[ch] * onehot, axis=1, keepdims=True)
                for ch in range(5)]

    def block_body(m, _):
        tx1, ty1, tx2, ty2, ta = cols_of(m)

        def k_body(k, hit_col):
            def c_body(c, hit_col):
                slot = k * _NC + c
                x1s = comp_ref[0, pl.ds(slot, 1)].reshape(1, _BC)
                y1s = comp_ref[1, pl.ds(slot, 1)].reshape(1, _BC)
                x2s = comp_ref[2, pl.ds(slot, 1)].reshape(1, _BC)
                y2s = comp_ref[3, pl.ds(slot, 1)].reshape(1, _BC)
                as_ = comp_ref[4, pl.ds(slot, 1)].reshape(1, _BC)
                cxx1 = jnp.maximum(x1s, tx1)
                cyy1 = jnp.maximum(y1s, ty1)
                cxx2 = jnp.minimum(x2s, tx2)
                cyy2 = jnp.minimum(y2s, ty2)
                cinter = (jnp.maximum(cxx2 - cxx1, 0.0)
                          * jnp.maximum(cyy2 - cyy1, 0.0))
                ciou = cinter / (as_ + ta - cinter + 1e-9)
                return jnp.maximum(
                    hit_col,
                    jnp.max((ciou > iou_thr).astype(jnp.float32),
                            axis=1, keepdims=True))

            return lax.fori_loop(0, nch_ref[k], c_body, hit_col)

        hit_col = lax.fori_loop(0, m, k_body,
                                jnp.zeros((_B, 1), jnp.float32))
        hit_row = (lax.dot_general(
            hit_col, ident, (((0,), (0,)), ((), ())),
            preferred_element_type=jnp.float32) > 0.5).astype(jnp.float32)

        x1r = d_ref[0, pl.ds(m, 1)].reshape(1, _B)
        y1r = d_ref[1, pl.ds(m, 1)].reshape(1, _B)
        x2r = d_ref[2, pl.ds(m, 1)].reshape(1, _B)
        y2r = d_ref[3, pl.ds(m, 1)].reshape(1, _B)
        ar = d_ref[4, pl.ds(m, 1)].reshape(1, _B)
        xx1 = jnp.maximum(tx1, x1r)
        yy1 = jnp.maximum(ty1, y1r)
        xx2 = jnp.minimum(tx2, x2r)
        yy2 = jnp.minimum(ty2, y2r)
        inter = jnp.maximum(xx2 - xx1, 0.0) * jnp.maximum(yy2 - yy1, 0.0)
        iou = inter / (ta + ar - inter + 1e-9)
        m_mat = (iou > iou_thr).astype(jnp.float32) * tri

        dead0 = jnp.maximum(sup_ref[pl.ds(m, 1)].reshape(1, _B), hit_row)

        def fix_cond(c):
            dead, alive, r = c
            return jnp.logical_and(
                jnp.sum((1.0 - dead) * (1.0 - alive)) > 0.0, r < _B)

        def fix_body(c):
            dead, alive, r = c
            notdead = 1.0 - dead
            maybe = jnp.dot(notdead, m_mat, preferred_element_type=jnp.float32)
            alive = jnp.maximum(
                alive, notdead * (maybe <= 0.0).astype(jnp.float32))
            defs = jnp.dot(alive, m_mat, preferred_element_type=jnp.float32)
            dead = jnp.maximum(
                dead, (defs > 0.0).astype(jnp.float32) * (1.0 - alive))
            return dead, alive, r + 1

        _, alive, _ = lax.while_loop(
            fix_cond, fix_body,
            (dead0, jnp.zeros((1, _B), jnp.float32), jnp.int32(0)))
        sup_ref[pl.ds(m, 1)] = (1.0 - alive).reshape(1, 1, _B)

        a_cnt = jnp.sum(alive).astype(jnp.int32)
        n_chunks = (a_cnt + _BC - 1) // _BC
        nch_ref[m] = n_chunks

        aliveT = (lax.dot_general(
            ident, alive, (((1,), (1,)), ((), ())),
            preferred_element_type=jnp.float32) > 0.5).astype(jnp.float32)
        pos_col = jnp.dot(lowtri, aliveT, preferred_element_type=jnp.float32)

        def store_chunk(c, _):
            qoff = qf + (c * _BC).astype(jnp.float32)
            perm = ((jnp.abs(pos_col - qoff) < 0.5).astype(jnp.float32)
                    * aliveT)
            slot = m * _NC + c
            for ch, col in enumerate((tx1, ty1, tx2, ty2, ta)):
                comp_ref[ch, pl.ds(slot, 1)] = jnp.sum(
                    perm * col, axis=0, keepdims=True).reshape(1, 1, _BC)
            return 0

        lax.fori_loop(0, n_chunks, store_chunk, 0)
        return 0

    lax.fori_loop(0, k_eff, block_body, 0)

    keep = 1.0 - sup_ref[...]
    out_ref[0] = d_ref[0] * keep
    out_ref[1] = d_ref[1] * keep
    out_ref[2] = d_ref[2] * keep
    out_ref[3] = d_ref[3] * keep
    out_ref[4] = s * keep


def _nms_pallas(ys_pad, scal, interpret=False):
    yolo_in = ys_pad.reshape(_K, 1, _B, 5).transpose(3, 0, 1, 2)
    yoloT_in = jnp.pad(
        ys_pad[:, :4].reshape(_K, _B, 4).transpose(2, 1, 0),
        ((0, 0), (0, 0), (0, _KL - _K)))
    out = pl.pallas_call(
        _nms_body,
        out_shape=jax.ShapeDtypeStruct((5, _K, 1, _B), jnp.float32),
        in_specs=[
            pl.BlockSpec(memory_space=pltpu.SMEM),
            pl.BlockSpec(memory_space=pltpu.VMEM),
            pl.BlockSpec(memory_space=pltpu.VMEM),
        ],
        out_specs=pl.BlockSpec(memory_space=pltpu.VMEM),
        scratch_shapes=[
            pltpu.VMEM((5, _K, 1, _B), jnp.float32),
            pltpu.VMEM((5, _B, _KL), jnp.float32),
            pltpu.VMEM((_K, 1, _B), jnp.float32),
            pltpu.VMEM((5, _K * _NC, 1, _BC), jnp.float32),
            pltpu.SMEM((_K,), jnp.int32),
        ],
        interpret=interpret,
    )(scal, yolo_in, yoloT_in)
    return out.reshape(5, _NP).T[:_N]


_GD = 128


def _sc_gather(table16, idx_pad, gb, b_per_w, num_cores):
    mesh = plsc.VectorSubcoreMesh(core_axis_name="c", subcore_axis_name="s")

    @functools.partial(
        pl.kernel, mesh=mesh,
        out_type=jax.ShapeDtypeStruct((gb, _GD), jnp.float32),
        scratch_types=[
            pltpu.VMEM((b_per_w,), jnp.int32),
            pltpu.VMEM((b_per_w, _GD), jnp.float32),
            pltpu.SemaphoreType.DMA,
        ],
    )
    def gk(table_hbm, idx_hbm, out_hbm, idx_v, rows_v, sem):
        wid = lax.axis_index("s") * num_cores + lax.axis_index("c")
        base = wid * b_per_w
        pltpu.sync_copy(idx_hbm.at[pl.ds(base, b_per_w)], idx_v)
        pltpu.async_copy(table_hbm.at[idx_v], rows_v, sem).wait()
        pltpu.sync_copy(rows_v, out_hbm.at[pl.ds(base, b_per_w)])

    return gk(table16, idx_pad)


def kernel(yolo_results, iou_threshold, score_threshold):
    scores = yolo_results[:, 4]
    order = jnp.argsort(-scores)
    info = plsc.get_sparse_core_info()
    nw = info.num_cores * info.num_subcores
    gb = ((_N + 8 * nw - 1) // (8 * nw)) * (8 * nw)
    idx_pad = jnp.concatenate(
        [order.astype(jnp.int32), jnp.zeros((gb - _N,), jnp.int32)])
    table16 = jnp.pad(yolo_results, ((0, 0), (0, _GD - 5)))
    ys = _sc_gather(table16, idx_pad, gb, gb // nw, info.num_cores)[:_N, :5]
    pad_row = jnp.array([[0.0, 0.0, 0.0, 0.0, -1.0]], jnp.float32)
    ys_pad = jnp.concatenate(
        [ys, jnp.broadcast_to(pad_row, (_NP - _N, 5))], axis=0)
    scal = jnp.stack([jnp.float32(iou_threshold), jnp.float32(score_threshold)])
    return _nms_pallas(ys_pad, scal)


# speedup vs baseline: 1.1334x; 1.1334x over previous
import functools

import jax
import jax.numpy as jnp
from jax import lax
from jax.experimental import pallas as pl
from jax.experimental.pallas import tpu as pltpu
from jax.experimental.pallas import tpu_sc as plsc

_N = 20000
_B = 1024
_K = 20
_NP = _B * _K
_KL = 128
_BC = 256
_SLOTS = _NP // _BC


def _nms_body(scal_ref, yolo_ref, yoloT_ref, out_ref, d_ref, dcol_ref,
              sup_ref, comp_ref, cnt_ref):
    iou_thr = scal_ref[0]
    score_thr = scal_ref[1]
    comp_ref[...] = jnp.zeros((5, _SLOTS, 1, _BC), jnp.float32)
    cnt_ref[0] = 0

    cx = yolo_ref[0]
    cy = yolo_ref[1]
    w = yolo_ref[2]
    h = yolo_ref[3]
    s = yolo_ref[4]
    x1 = cx - w / 2
    y1 = cy - h / 2
    x2 = cx + w / 2
    y2 = cy + h / 2
    area = jnp.maximum(x2 - x1, 0.0) * jnp.maximum(y2 - y1, 0.0)
    d_ref[0] = x1
    d_ref[1] = y1
    d_ref[2] = x2
    d_ref[3] = y2
    d_ref[4] = area

    cxT = yoloT_ref[0]
    cyT = yoloT_ref[1]
    wT = yoloT_ref[2]
    hT = yoloT_ref[3]
    x1T = cxT - wT / 2
    y1T = cyT - hT / 2
    x2T = cxT + wT / 2
    y2T = cyT + hT / 2
    dcol_ref[0] = x1T
    dcol_ref[1] = y1T
    dcol_ref[2] = x2T
    dcol_ref[3] = y2T
    dcol_ref[4] = jnp.maximum(x2T - x1T, 0.0) * jnp.maximum(y2T - y1T, 0.0)

    sup_ref[...] = (s <= score_thr).astype(jnp.float32)

    n_eff = jnp.sum((s > score_thr).astype(jnp.int32))
    k_eff = (n_eff + _B - 1) // _B

    ri = lax.broadcasted_iota(jnp.int32, (_B, _B), 0)
    ci = lax.broadcasted_iota(jnp.int32, (_B, _B), 1)
    tri = (ci > ri).astype(jnp.float32)
    ident = (ci == ri).astype(jnp.float32)
    lowtri = (ci < ri).astype(jnp.float32)
    qf = lax.broadcasted_iota(jnp.int32, (_B, _BC), 1).astype(jnp.float32)
    lane_k = lax.broadcasted_iota(jnp.int32, (1, _KL), 1)

    def cols_of(m):
        onehot = (lane_k == m).astype(jnp.float32)
        return [jnp.sum(dcol_ref[ch] * onehot, axis=1, keepdims=True)
                for ch in range(5)]

    def block_body(m, _):
        tx1, ty1, tx2, ty2, ta = cols_of(m)

        g_cnt = cnt_ref[0]

        def c_body(c, hit_col):
            x1s = comp_ref[0, pl.ds(c, 1)].reshape(1, _BC)
            y1s = comp_ref[1, pl.ds(c, 1)].reshape(1, _BC)
            x2s = comp_ref[2, pl.ds(c, 1)].reshape(1, _BC)
            y2s = comp_ref[3, pl.ds(c, 1)].reshape(1, _BC)
            as_ = comp_ref[4, pl.ds(c, 1)].reshape(1, _BC)
            cxx1 = jnp.maximum(x1s, tx1)
            cyy1 = jnp.maximum(y1s, ty1)
            cxx2 = jnp.minimum(x2s, tx2)
            cyy2 = jnp.minimum(y2s, ty2)
            cinter = (jnp.maximum(cxx2 - cxx1, 0.0)
                      * jnp.maximum(cyy2 - cyy1, 0.0))
            ciou = cinter / (as_ + ta - cinter + 1e-9)
            return jnp.maximum(
                hit_col,
                jnp.max((ciou > iou_thr).astype(jnp.float32),
                        axis=1, keepdims=True))

        hit_col = lax.fori_loop(0, (g_cnt + _BC - 1) // _BC, c_body,
                                jnp.zeros((_B, 1), jnp.float32))
        hit_row = (lax.dot_general(
            hit_col, ident, (((0,), (0,)), ((), ())),
            preferred_element_type=jnp.float32) > 0.5).astype(jnp.float32)

        x1r = d_ref[0, pl.ds(m, 1)].reshape(1, _B)
        y1r = d_ref[1, pl.ds(m, 1)].reshape(1, _B)
        x2r = d_ref[2, pl.ds(m, 1)].reshape(1, _B)
        y2r = d_ref[3, pl.ds(m, 1)].reshape(1, _B)
        ar = d_ref[4, pl.ds(m, 1)].reshape(1, _B)
        xx1 = jnp.maximum(tx1, x1r)
        yy1 = jnp.maximum(ty1, y1r)
        xx2 = jnp.minimum(tx2, x2r)
        yy2 = jnp.minimum(ty2, y2r)
        inter = jnp.maximum(xx2 - xx1, 0.0) * jnp.maximum(yy2 - yy1, 0.0)
        iou = inter / (ta + ar - inter + 1e-9)
        m_mat = (iou > iou_thr).astype(jnp.float32) * tri

        dead0 = jnp.maximum(sup_ref[pl.ds(m, 1)].reshape(1, _B), hit_row)

        def fix_cond(c):
            dead, alive, r = c
            return jnp.logical_and(
                jnp.sum((1.0 - dead) * (1.0 - alive)) > 0.0, r < _B)

        def fix_body(c):
            dead, alive, r = c
            notdead = 1.0 - dead
            maybe = jnp.dot(notdead, m_mat, preferred_element_type=jnp.float32)
            alive = jnp.maximum(
                alive, notdead * (maybe <= 0.0).astype(jnp.float32))
            defs = jnp.dot(alive, m_mat, preferred_element_type=jnp.float32)
            dead = jnp.maximum(
                dead, (defs > 0.0).astype(jnp.float32) * (1.0 - alive))
            return dead, alive, r + 1

        _, alive, _ = lax.while_loop(
            fix_cond, fix_body,
            (dead0, jnp.zeros((1, _B), jnp.float32), jnp.int32(0)))
        sup_ref[pl.ds(m, 1)] = (1.0 - alive).reshape(1, 1, _B)

        a_cnt = jnp.sum(alive).astype(jnp.int32)

        aliveT = (lax.dot_general(
            ident, alive, (((1,), (1,)), ((), ())),
            preferred_element_type=jnp.float32) > 0.5).astype(jnp.float32)
        pos_col = (jnp.dot(lowtri, aliveT, preferred_element_type=jnp.float32)
                   + g_cnt.astype(jnp.float32))

        def store_chunk(c, _):
            qoff = qf + (c * _BC).astype(jnp.float32)
            perm = ((jnp.abs(pos_col - qoff) < 0.5).astype(jnp.float32)
                    * aliveT)
            for ch, col in enumerate((tx1, ty1, tx2, ty2, ta)):
                comp_ref[ch, pl.ds(c, 1)] = (
                    comp_ref[ch, pl.ds(c, 1)]
                    + jnp.sum(perm * col, axis=0,
                              keepdims=True).reshape(1, 1, _BC))
            return 0

        lax.fori_loop(g_cnt // _BC, (g_cnt + a_cnt + _BC - 1) // _BC,
                      store_chunk, 0)
        cnt_ref[0] = g_cnt + a_cnt
        return 0

    lax.fori_loop(0, k_eff, block_body, 0)

    keep = 1.0 - sup_ref[...]
    out_ref[0] = d_ref[0] * keep
    out_ref[1] = d_ref[1] * keep
    out_ref[2] = d_ref[2] * keep
    out_ref[3] = d_ref[3] * keep
    out_ref[4] = s * keep


def _nms_pallas(ys_pad, scal, interpret=False):
    yolo_in = ys_pad.reshape(_K, 1, _B, 5).transpose(3, 0, 1, 2)
    yoloT_in = jnp.pad(
        ys_pad[:, :4].reshape(_K, _B, 4).transpose(2, 1, 0),
        ((0, 0), (0, 0), (0, _KL - _K)))
    out = pl.pallas_call(
        _nms_body,
        out_shape=jax.ShapeDtypeStruct((5, _K, 1, _B), jnp.float32),
        in_specs=[
            pl.BlockSpec(memory_space=pltpu.SMEM),
            pl.BlockSpec(memory_space=pltpu.VMEM),
            pl.BlockSpec(memory_space=pltpu.VMEM),
        ],
        out_specs=pl.BlockSpec(memory_space=pltpu.VMEM),
        scratch_shapes=[
            pltpu.VMEM((5, _K, 1, _B), jnp.float32),
            pltpu.VMEM((5, _B, _KL), jnp.float32),
            pltpu.VMEM((_K, 1, _B), jnp.float32),
            pltpu.VMEM((5, _SLOTS, 1, _BC), jnp.float32),
            pltpu.SMEM((1,), jnp.int32),
        ],
        interpret=interpret,
    )(scal, yolo_in, yoloT_in)
    return out.reshape(5, _NP).T[:_N]


_GD = 128


def _sc_gather(table16, idx_pad, gb, b_per_w, num_cores):
    mesh = plsc.VectorSubcoreMesh(core_axis_name="c", subcore_axis_name="s")

    @functools.partial(
        pl.kernel, mesh=mesh,
        out_type=jax.ShapeDtypeStruct((gb, _GD), jnp.float32),
        scratch_types=[
            pltpu.VMEM((b_per_w,), jnp.int32),
            pltpu.VMEM((b_per_w, _GD), jnp.float32),
            pltpu.SemaphoreType.DMA,
        ],
    )
    def gk(table_hbm, idx_hbm, out_hbm, idx_v, rows_v, sem):
        wid = lax.axis_index("s") * num_cores + lax.axis_index("c")
        base = wid * b_per_w
        pltpu.sync_copy(idx_hbm.at[pl.ds(base, b_per_w)], idx_v)
        pltpu.async_copy(table_hbm.at[idx_v], rows_v, sem).wait()
        pltpu.sync_copy(rows_v, out_hbm.at[pl.ds(base, b_per_w)])

    return gk(table16, idx_pad)


def kernel(yolo_results, iou_threshold, score_threshold):
    scores = yolo_results[:, 4]
    order = jnp.argsort(-scores)
    info = plsc.get_sparse_core_info()
    nw = info.num_cores * info.num_subcores
    gb = ((_N + 8 * nw - 1) // (8 * nw)) * (8 * nw)
    idx_pad = jnp.concatenate(
        [order.astype(jnp.int32), jnp.zeros((gb - _N,), jnp.int32)])
    table16 = jnp.pad(yolo_results, ((0, 0), (0, _GD - 5)))
    ys = _sc_gather(table16, idx_pad, gb, gb // nw, info.num_cores)[:_N, :5]
    pad_row = jnp.array([[0.0, 0.0, 0.0, 0.0, -1.0]], jnp.float32)
    ys_pad = jnp.concatenate(
        [ys, jnp.broadcast_to(pad_row, (_NP - _N, 5))], axis=0)
    scal = jnp.stack([jnp.float32(iou_threshold), jnp.float32(score_threshold)])
    return _nms_pallas(ys_pad, scal)
